# baseline (speedup 1.0000x reference)
import jax
import jax.numpy as jnp
from jax import lax
from jax.experimental import pallas as pl
from jax.experimental.pallas import tpu as pltpu
from jax.experimental.pallas import tpu_sc as plsc

_B = 524288
_D = 16
_IGNORE = -1000.0
_BIG = float(jnp.finfo(jnp.float32).max)
_ABS_HI = 0x7FFFFFF0
_HI_MASK = -16

_NC, _NS, _L = 2, 16, 16
_NW = _NC * _NS

_BS = 229376
_BT = _B - _BS
_RW = _BS // _NW
_C = 256
_NCHUNK = _RW // _C
_G = _C // _L

_TCB = 8192
_TCN = _BT // _TCB


def _body(out_hbm, tgt_hbm, loss_hbm, obuf, tbuf, accbuf,
          sem_o0, sem_o1, sem_t0, sem_t1):
  wid = lax.axis_index("s") * _NC + lax.axis_index("c")
  base = wid * _RW
  kconsts = [jnp.full((_L,), k, jnp.int32) for k in range(_D)]
  sems_o = (sem_o0, sem_o1)
  sems_t = (sem_t0, sem_t1)

  def start_copies(b, start):
    pltpu.async_copy(out_hbm.at[:, pl.ds(start, _C)], obuf.at[b], sems_o[b])
    pltpu.async_copy(tgt_hbm.at[:, pl.ds(start, _C)], tbuf.at[b], sems_t[b])

  def wait_copies(b):
    pltpu.make_async_copy(
        out_hbm.at[:, pl.ds(0, _C)], obuf.at[b], sems_o[b]).wait()
    pltpu.make_async_copy(
        tgt_hbm.at[:, pl.ds(0, _C)], tbuf.at[b], sems_t[b]).wait()

  def make_group(b):
    def group(g, acc):
      gb = g * _L
      cur = [obuf[b, k, pl.ds(gb, _L)] for k in range(_D)]

      for i in range(_D - 1):
        t = tbuf[b, i, pl.ds(gb, _L)]
        dp = [(lax.bitcast_convert_type(cur[k] - t, jnp.int32) & _ABS_HI) | k
              for k in range(_D)]
        n = _D
        while n > 1:
          half = n // 2
          for a in range(half):
            dp[a] = jnp.minimum(dp[a], dp[a + half])
          n = half
        dwin = dp[0]
        idx = dwin & 0xF
        dmin = lax.bitcast_convert_type(dwin & _HI_MASK, jnp.float32)
        cur = [jnp.where(idx == kconsts[k], _BIG, cur[k])
               for k in range(_D)]
        acc = acc + jnp.where(t != _IGNORE, dmin * dmin, 0.0)

      t = tbuf[b, _D - 1, pl.ds(gb, _L)]
      vs = list(cur)
      n = _D
      while n > 1:
        half = n // 2
        for a in range(half):
          vs[a] = jnp.minimum(vs[a], vs[a + half])
        n = half
      diff = vs[0] - t
      acc = acc + jnp.where(t != _IGNORE, diff * diff, 0.0)
      return acc

    return group

  groups = (make_group(0), make_group(1))

  start_copies(0, base)
  start_copies(1, base + _C)

  def chunk_pair(c2, acc):
    for b in range(2):
      ci = c2 * 2 + b
      wait_copies(b)
      acc = lax.fori_loop(0, _G, groups[b], acc)
      nstart = jnp.minimum(base + (ci + 2) * _C, _B - _C)
      start_copies(b, nstart)
    return acc

  acc = lax.fori_loop(0, _NCHUNK // 2, chunk_pair,
                      jnp.zeros((_L,), jnp.float32))
  wait_copies(0)
  wait_copies(1)
  accbuf[...] = acc
  pltpu.sync_copy(accbuf, loss_hbm.at[wid])


def _make_call():
  mesh = plsc.VectorSubcoreMesh(core_axis_name="c", subcore_axis_name="s")
  return pl.kernel(
      _body,
      out_type=jax.ShapeDtypeStruct((_NW, _L), jnp.float32),
      mesh=mesh,
      compiler_params=pltpu.CompilerParams(
          needs_layout_passes=False,
          use_tc_tiling_on_sc=True,
      ),
      scratch_types=[
          pltpu.VMEM((2, _D, _C), jnp.float32),
          pltpu.VMEM((2, _D, _C), jnp.float32),
          pltpu.VMEM((_L,), jnp.float32),
          pltpu.SemaphoreType.DMA,
          pltpu.SemaphoreType.DMA,
          pltpu.SemaphoreType.DMA,
          pltpu.SemaphoreType.DMA,
      ],
  )


def _tc_body(o_ref, t_ref, loss_ref):
  j = pl.program_id(0)
  iota = lax.broadcasted_iota(jnp.int32, (_D, _TCB), 0)
  cur = o_ref[...]
  acc = jnp.zeros((1, _TCB), jnp.float32)

  for i in range(_D):
    t = t_ref[i:i + 1, :]
    if i < _D - 1:
      dp = (lax.bitcast_convert_type(cur - t, jnp.int32) & _ABS_HI) | iota
      dwin = lax.bitcast_convert_type(
          jnp.min(lax.bitcast_convert_type(dp, jnp.float32),
                  axis=0, keepdims=True),
          jnp.int32)
      onehot = dp == dwin
      cur = jnp.where(onehot, _BIG, cur)
      dmin = lax.bitcast_convert_type(dwin & _HI_MASK, jnp.float32)
      acc = acc + jnp.where(t != _IGNORE, dmin * dmin, 0.0)
    else:
      val = jnp.min(cur, axis=0, keepdims=True)
      diff = val - t
      acc = acc + jnp.where(t != _IGNORE, diff * diff, 0.0)

  @pl.when(j == 0)
  def _():
    loss_ref[...] = jnp.zeros_like(loss_ref)

  loss_ref[...] += acc


def _make_tc_call():
  return pl.pallas_call(
      _tc_body,
      grid=(_TCN,),
      in_specs=[
          pl.BlockSpec((_D, _TCB), lambda j: (0, _BS // _TCB + j)),
          pl.BlockSpec((_D, _TCB), lambda j: (0, _BS // _TCB + j)),
      ],
      out_specs=pl.BlockSpec((1, _TCB), lambda j: (0, 0)),
      out_shape=jax.ShapeDtypeStruct((1, _TCB), jnp.float32),
      compiler_params=pltpu.CompilerParams(
          dimension_semantics=("arbitrary",)),
  )


@jax.jit
def kernel(outputs, targets):
  ot, tt = outputs.T, targets.T
  sc_partial = _make_call()(ot, tt)
  tc_partial = _make_tc_call()(ot, tt)
  return (jnp.sum(sc_partial) + jnp.sum(tc_partial)) / (_B * _D)


# speedup vs baseline: 1.0727x; 1.0727x over previous
import jax
import jax.numpy as jnp
from jax import lax
from jax.experimental import pallas as pl
from jax.experimental.pallas import tpu as pltpu
from jax.experimental.pallas import tpu_sc as plsc

_B = 524288
_D = 16
_IGNORE = -1000.0
_BIG = float(jnp.finfo(jnp.float32).max)
_ABS_HI = 0x7FFFFFF0
_HI_MASK = -16

_NC, _NS, _L = 2, 16, 16
_NW = _NC * _NS

_BS = 229376
_BT = _B - _BS
_RW = _BS // _NW
_C = 512
_NCHUNK = _RW // _C
_G = _C // _L

_TCB = 4096
_TCN = _BT // _TCB


def _body(out_hbm, tgt_hbm, loss_hbm, obuf, tbuf, accbuf,
          sem_o0, sem_o1, sem_t0, sem_t1):
  wid = lax.axis_index("s") * _NC + lax.axis_index("c")
  base = wid * _RW
  kconsts = [jnp.full((_L,), k, jnp.int32) for k in range(_D)]
  sems_o = (sem_o0, sem_o1)
  sems_t = (sem_t0, sem_t1)

  def start_copies(b, start):
    pltpu.async_copy(out_hbm.at[:, pl.ds(start, _C)], obuf.at[b], sems_o[b])
    pltpu.async_copy(tgt_hbm.at[:, pl.ds(start, _C)], tbuf.at[b], sems_t[b])

  def wait_copies(b):
    pltpu.make_async_copy(
        out_hbm.at[:, pl.ds(0, _C)], obuf.at[b], sems_o[b]).wait()
    pltpu.make_async_copy(
        tgt_hbm.at[:, pl.ds(0, _C)], tbuf.at[b], sems_t[b]).wait()

  def make_group(b):
    def group(g, acc):
      gb = g * _L
      cur = [obuf[b, k, pl.ds(gb, _L)] for k in range(_D)]

      for i in range(_D - 1):
        t = tbuf[b, i, pl.ds(gb, _L)]
        dp = [(lax.bitcast_convert_type(cur[k] - t, jnp.int32) & _ABS_HI) | k
              for k in range(_D)]
        n = _D
        while n > 1:
          half = n // 2
          for a in range(half):
            dp[a] = jnp.minimum(dp[a], dp[a + half])
          n = half
        dwin = dp[0]
        idx = dwin & 0xF
        dmin = lax.bitcast_convert_type(dwin & _HI_MASK, jnp.float32)
        cur = [jnp.where(idx == kconsts[k], _BIG, cur[k])
               for k in range(_D)]
        acc = acc + jnp.where(t != _IGNORE, dmin * dmin, 0.0)

      t = tbuf[b, _D - 1, pl.ds(gb, _L)]
      vs = list(cur)
      n = _D
      while n > 1:
        half = n // 2
        for a in range(half):
          vs[a] = jnp.minimum(vs[a], vs[a + half])
        n = half
      diff = vs[0] - t
      acc = acc + jnp.where(t != _IGNORE, diff * diff, 0.0)
      return acc

    return group

  groups = (make_group(0), make_group(1))

  start_copies(0, base)
  start_copies(1, base + _C)

  def chunk_pair(c2, acc):
    for b in range(2):
      ci = c2 * 2 + b
      wait_copies(b)
      acc = lax.fori_loop(0, _G, groups[b], acc)
      nstart = jnp.minimum(base + (ci + 2) * _C, _B - _C)
      start_copies(b, nstart)
    return acc

  acc = lax.fori_loop(0, _NCHUNK // 2, chunk_pair,
                      jnp.zeros((_L,), jnp.float32))
  wait_copies(0)
  wait_copies(1)
  accbuf[...] = acc
  pltpu.sync_copy(accbuf, loss_hbm.at[wid])


def _make_call():
  mesh = plsc.VectorSubcoreMesh(core_axis_name="c", subcore_axis_name="s")
  return pl.kernel(
      _body,
      out_type=jax.ShapeDtypeStruct((_NW, _L), jnp.float32),
      mesh=mesh,
      compiler_params=pltpu.CompilerParams(
          needs_layout_passes=False,
          use_tc_tiling_on_sc=True,
      ),
      scratch_types=[
          pltpu.VMEM((2, _D, _C), jnp.float32),
          pltpu.VMEM((2, _D, _C), jnp.float32),
          pltpu.VMEM((_L,), jnp.float32),
          pltpu.SemaphoreType.DMA,
          pltpu.SemaphoreType.DMA,
          pltpu.SemaphoreType.DMA,
          pltpu.SemaphoreType.DMA,
      ],
  )


def _tc_body(o_ref, t_ref, loss_ref):
  j = pl.program_id(0)
  iota = lax.broadcasted_iota(jnp.int32, (_D, _TCB), 0)
  cur = o_ref[...]
  acc = jnp.zeros((1, _TCB), jnp.float32)

  for i in range(_D):
    t = t_ref[i:i + 1, :]
    if i < _D - 1:
      dp = (lax.bitcast_convert_type(cur - t, jnp.int32) & _ABS_HI) | iota
      dwin = lax.bitcast_convert_type(
          jnp.min(lax.bitcast_convert_type(dp, jnp.float32),
                  axis=0, keepdims=True),
          jnp.int32)
      onehot = dp == dwin
      cur = jnp.where(onehot, _BIG, cur)
      dmin = lax.bitcast_convert_type(dwin & _HI_MASK, jnp.float32)
      acc = acc + jnp.where(t != _IGNORE, dmin * dmin, 0.0)
    else:
      val = jnp.min(cur, axis=0, keepdims=True)
      diff = val - t
      acc = acc + jnp.where(t != _IGNORE, diff * diff, 0.0)

  @pl.when(j == 0)
  def _():
    loss_ref[...] = jnp.zeros_like(loss_ref)

  loss_ref[...] += acc


def _make_tc_call():
  return pl.pallas_call(
      _tc_body,
      grid=(_TCN,),
      in_specs=[
          pl.BlockSpec((_D, _TCB), lambda j: (0, _BS // _TCB + j)),
          pl.BlockSpec((_D, _TCB), lambda j: (0, _BS // _TCB + j)),
      ],
      out_specs=pl.BlockSpec((1, _TCB), lambda j: (0, 0)),
      out_shape=jax.ShapeDtypeStruct((1, _TCB), jnp.float32),
      compiler_params=pltpu.CompilerParams(
          dimension_semantics=("arbitrary",)),
  )


@jax.jit
def kernel(outputs, targets):
  ot, tt = outputs.T, targets.T
  sc_partial = _make_call()(ot, tt)
  tc_partial = _make_tc_call()(ot, tt)
  return (jnp.sum(sc_partial) + jnp.sum(tc_partial)) / (_B * _D)


# speedup vs baseline: 1.0791x; 1.0060x over previous
import jax
import jax.numpy as jnp
from jax import lax
from jax.experimental import pallas as pl
from jax.experimental.pallas import tpu as pltpu
from jax.experimental.pallas import tpu_sc as plsc

_B = 524288
_D = 16
_IGNORE = -1000.0
_BIG = float(jnp.finfo(jnp.float32).max)
_ABS_HI = 0x7FFFFFF0
_HI_MASK = -16

_NC, _NS, _L = 2, 16, 16
_NW = _NC * _NS

_BS = 229376
_BT = _B - _BS
_RW = _BS // _NW
_C = 256
_NCHUNK = _RW // _C
_G = _C // _L

_TCB = 4096
_TCN = _BT // _TCB


def _body(out_hbm, tgt_hbm, loss_hbm, obuf, tbuf, accbuf,
          sem_o0, sem_o1, sem_t0, sem_t1):
  wid = lax.axis_index("s") * _NC + lax.axis_index("c")
  base = wid * _RW
  kconsts = [jnp.full((_L,), k, jnp.int32) for k in range(_D)]
  sems_o = (sem_o0, sem_o1)
  sems_t = (sem_t0, sem_t1)

  def start_copies(b, start):
    pltpu.async_copy(out_hbm.at[:, pl.ds(start, _C)], obuf.at[b], sems_o[b])
    pltpu.async_copy(tgt_hbm.at[:, pl.ds(start, _C)], tbuf.at[b], sems_t[b])

  def wait_copies(b):
    pltpu.make_async_copy(
        out_hbm.at[:, pl.ds(0, _C)], obuf.at[b], sems_o[b]).wait()
    pltpu.make_async_copy(
        tgt_hbm.at[:, pl.ds(0, _C)], tbuf.at[b], sems_t[b]).wait()

  def make_group(b):
    def group(g, acc):
      gb = g * _L
      cur = [obuf[b, k, pl.ds(gb, _L)] for k in range(_D)]

      for i in range(_D - 1):
        t = tbuf[b, i, pl.ds(gb, _L)]
        dp = [(lax.bitcast_convert_type(cur[k] - t, jnp.int32) & _ABS_HI) | k
              for k in range(_D)]
        n = _D
        while n > 1:
          half = n // 2
          for a in range(half):
            dp[a] = jnp.minimum(dp[a], dp[a + half])
          n = half
        dwin = dp[0]
        idx = dwin & 0xF
        dmin = lax.bitcast_convert_type(dwin & _HI_MASK, jnp.float32)
        cur = [jnp.where(idx == kconsts[k], _BIG, cur[k])
               for k in range(_D)]
        acc = acc + jnp.where(t != _IGNORE, dmin * dmin, 0.0)

      t = tbuf[b, _D - 1, pl.ds(gb, _L)]
      vs = list(cur)
      n = _D
      while n > 1:
        half = n // 2
        for a in range(half):
          vs[a] = jnp.minimum(vs[a], vs[a + half])
        n = half
      diff = vs[0] - t
      acc = acc + jnp.where(t != _IGNORE, diff * diff, 0.0)
      return acc

    return group

  groups = (make_group(0), make_group(1))

  start_copies(0, base)
  start_copies(1, base + _C)

  def chunk_pair(c2, acc):
    for b in range(2):
      ci = c2 * 2 + b
      wait_copies(b)
      acc = lax.fori_loop(0, _G, groups[b], acc)
      nstart = jnp.minimum(base + (ci + 2) * _C, _B - _C)
      start_copies(b, nstart)
    return acc

  acc = lax.fori_loop(0, _NCHUNK // 2, chunk_pair,
                      jnp.zeros((_L,), jnp.float32))
  wait_copies(0)
  wait_copies(1)
  accbuf[...] = acc
  pltpu.sync_copy(accbuf, loss_hbm.at[wid])


def _make_call():
  mesh = plsc.VectorSubcoreMesh(core_axis_name="c", subcore_axis_name="s")
  return pl.kernel(
      _body,
      out_type=jax.ShapeDtypeStruct((_NW, _L), jnp.float32),
      mesh=mesh,
      compiler_params=pltpu.CompilerParams(
          needs_layout_passes=False,
          use_tc_tiling_on_sc=True,
      ),
      scratch_types=[
          pltpu.VMEM((2, _D, _C), jnp.float32),
          pltpu.VMEM((2, _D, _C), jnp.float32),
          pltpu.VMEM((_L,), jnp.float32),
          pltpu.SemaphoreType.DMA,
          pltpu.SemaphoreType.DMA,
          pltpu.SemaphoreType.DMA,
          pltpu.SemaphoreType.DMA,
      ],
  )


def _tc_body(o_ref, t_ref, loss_ref):
  j = pl.program_id(0)
  iota = lax.broadcasted_iota(jnp.int32, (_D, _TCB), 0)
  cur = o_ref[...]
  acc = jnp.zeros((1, _TCB), jnp.float32)

  for i in range(_D):
    t = t_ref[i:i + 1, :]
    if i < _D - 1:
      dp = (lax.bitcast_convert_type(cur - t, jnp.int32) & _ABS_HI) | iota
      dwin = lax.bitcast_convert_type(
          jnp.min(lax.bitcast_convert_type(dp, jnp.float32),
                  axis=0, keepdims=True),
          jnp.int32)
      onehot = dp == dwin
      cur = jnp.where(onehot, _BIG, cur)
      dmin = lax.bitcast_convert_type(dwin & _HI_MASK, jnp.float32)
      acc = acc + jnp.where(t != _IGNORE, dmin * dmin, 0.0)
    else:
      val = jnp.min(cur, axis=0, keepdims=True)
      diff = val - t
      acc = acc + jnp.where(t != _IGNORE, diff * diff, 0.0)

  @pl.when(j == 0)
  def _():
    loss_ref[...] = jnp.zeros_like(loss_ref)

  loss_ref[...] += acc


def _make_tc_call():
  return pl.pallas_call(
      _tc_body,
      grid=(_TCN,),
      in_specs=[
          pl.BlockSpec((_D, _TCB), lambda j: (0, _BS // _TCB + j)),
          pl.BlockSpec((_D, _TCB), lambda j: (0, _BS // _TCB + j)),
      ],
      out_specs=pl.BlockSpec((1, _TCB), lambda j: (0, 0)),
      out_shape=jax.ShapeDtypeStruct((1, _TCB), jnp.float32),
      compiler_params=pltpu.CompilerParams(
          dimension_semantics=("arbitrary",)),
  )


@jax.jit
def kernel(outputs, targets):
  ot, tt = outputs.T, targets.T
  sc_partial = _make_call()(ot, tt)
  tc_partial = _make_tc_call()(ot, tt)
  return (jnp.sum(sc_partial) + jnp.sum(tc_partial)) / (_B * _D)
